# baseline (speedup 1.0000x reference)
import functools

import jax
import jax.numpy as jnp
import numpy as np
from jax.experimental import pallas as pl


N = 10000
D = 128
LAM_C = 0.65
LAM_R = 0.35
THETA0 = 0.5
SPEC_MIN = 0.3
SPEC_MAX = 4.0
HOPS = 3


def _pre_kernel(yh_ref, ys_ref, vin_ref, tout_ref, nsq_ref, nrm_ref):
    bq = yh_ref.shape[0]
    tout_ref[...] = jnp.sum((ys_ref[...] - yh_ref[...]), axis=0) / (
        bq * np.sqrt(D)
    )
    nsq = jnp.sum(vin_ref[...] * vin_ref[...], axis=1)
    nsq_ref[...] = nsq
    nrm_ref[...] = jnp.sqrt(nsq)


def _dense_kernel(vout_ref, vw_ref, good_ref, nsq_ref, a_ref, tout_ref, out_ref):
    theta = THETA0 * (nsq_ref[...] + 1e-9)
    delta_g = good_ref[...] - theta
    f_prime = 1.0 - jnp.tanh(vw_ref[...]) ** 2
    t_rows = a_ref[...][:, None] * tout_ref[...][None, :]
    dw = LAM_C * vout_ref[...] * f_prime * delta_g[:, None] + LAM_R * (
        t_rows * f_prime
    )
    g = jnp.sqrt(jnp.sum(dw * dw, axis=1, keepdims=True))
    out_ref[...] = jnp.where(g > 5.0, dw * (5.0 / (g + 1e-12)), dw)


def kernel(Y_hat, Y_star, V_in, V_out, V_weighted, goodness, spec_norm, edge_index):
    n = V_in.shape[0]
    tout, nsq, nrm = pl.pallas_call(
        _pre_kernel,
        out_shape=[
            jax.ShapeDtypeStruct((D,), jnp.float32),
            jax.ShapeDtypeStruct((n,), jnp.float32),
            jax.ShapeDtypeStruct((n,), jnp.float32),
        ],
    )(Y_hat, Y_star, V_in)

    src = edge_index[0]
    dst = edge_index[1]
    norms = nrm[src]
    total = jax.ops.segment_sum(norms, dst, num_segments=n) + 1e-9
    coef = norms / total[dst] * jnp.clip(spec_norm, SPEC_MIN, SPEC_MAX)[dst]
    a = jnp.zeros((n,), jnp.float32).at[n - 1].set(1.0)
    for _ in range(HOPS):
        a = a + jax.ops.segment_sum(coef * a[dst], src, num_segments=n)

    dw = pl.pallas_call(
        _dense_kernel,
        out_shape=jax.ShapeDtypeStruct((n, D), jnp.float32),
    )(V_out, V_weighted, goodness, nsq, a, tout)
    return dw


# speedup vs baseline: 56.9984x; 56.9984x over previous
import functools

import jax
import jax.numpy as jnp
import numpy as np
from jax import lax
from jax.experimental import pallas as pl
from jax.experimental.pallas import tpu as pltpu
from jax.experimental.pallas import tpu_sc as plsc


N = 10000
E = 320000
D = 128
LAM_C = 0.65
LAM_R = 0.35
THETA0 = 0.5
SPEC_MIN = 0.3
SPEC_MAX = 4.0
HOPS = 3

NC = 2
NS = 16
NW = NC * NS
EW = E // NW
RW = 80
NG = N // 16

_MESH = plsc.VectorSubcoreMesh(
    core_axis_name="c", subcore_axis_name="s", num_cores=NC, num_subcores=NS
)
_SC_PARAMS = pltpu.CompilerParams(needs_layout_passes=False)


def _worker_id():
    cid = lax.axis_index("c")
    sid = lax.axis_index("s")
    return cid, sid, cid * NS + sid


def _stage_sum2(a_hbm, b_hbm, buf_a, buf_b, bias=0.0):
    pltpu.sync_copy(a_hbm, buf_a)
    pltpu.sync_copy(b_hbm, buf_b)

    def body(i, _):
        s = pl.ds(i * 16, 16)
        buf_a[s] = buf_a[s] + buf_b[s] + bias
        return 0

    lax.fori_loop(0, NG, body, 0)


def _valid(j, k):
    return (j < EW // 128) | ((j == EW // 128) & (k < (EW % 128) // 16))


def _sc_total_body(nrm_hbm, edge_hbm, zvec_hbm, out_hbm,
                   nrm_loc, src_t, dst_t, val_t, acc):
    cid, sid, wid = _worker_id()
    pltpu.sync_copy(nrm_hbm, nrm_loc)
    pltpu.sync_copy(edge_hbm.at[0, wid], src_t)
    pltpu.sync_copy(edge_hbm.at[1, wid], dst_t)

    @pl.when(sid == 0)
    def _():
        pltpu.sync_copy(zvec_hbm, acc)

    def row(j, _):
        for k in range(8):
            s = pl.ds(k * 16, 16)
            idx = src_t[j, s]
            v = plsc.load_gather(nrm_loc, [idx])
            val_t[j, s] = v * _valid(j, k).astype(jnp.float32)
        return 0

    lax.fori_loop(0, RW, row, 0)
    plsc.subcore_barrier()

    def srow(j, _):
        pltpu.sync_copy(val_t.at[j], acc.at[dst_t.at[j]], add=True)
        return 0

    lax.fori_loop(0, RW, srow, 0)
    plsc.subcore_barrier()

    @pl.when(sid == 0)
    def _():
        pltpu.sync_copy(acc, out_hbm.at[cid])


_sc_total = pl.kernel(
    _sc_total_body,
    out_type=jax.ShapeDtypeStruct((NC, N), jnp.float32),
    mesh=_MESH,
    compiler_params=_SC_PARAMS,
    scratch_types=[
        pltpu.VMEM((N,), jnp.float32),
        pltpu.VMEM((RW, 128), jnp.int32),
        pltpu.VMEM((RW, 128), jnp.int32),
        pltpu.VMEM((RW, 128), jnp.float32),
        pltpu.VMEM_SHARED((N,), jnp.float32),
    ],
)


def _sc_coef_body(nrm_hbm, spec_hbm, tot2_hbm, edge_hbm, coef_hbm,
                  nrm_loc, t_loc, t_b, spec_loc, src_t, dst_t, coef_t):
    cid, sid, wid = _worker_id()
    pltpu.sync_copy(nrm_hbm, nrm_loc)
    pltpu.sync_copy(spec_hbm, spec_loc)
    _stage_sum2(tot2_hbm.at[0], tot2_hbm.at[1], t_loc, t_b, bias=1e-9)
    pltpu.sync_copy(edge_hbm.at[0, wid], src_t)
    pltpu.sync_copy(edge_hbm.at[1, wid], dst_t)

    def row(j, _):
        for k in range(8):
            s = pl.ds(k * 16, 16)
            si = src_t[j, s]
            di = dst_t[j, s]
            v = plsc.load_gather(nrm_loc, [si])
            t = plsc.load_gather(t_loc, [di])
            sp = plsc.load_gather(spec_loc, [di])
            sp = jnp.minimum(jnp.maximum(sp, SPEC_MIN), SPEC_MAX)
            coef_t[j, s] = v / t * sp
        return 0

    lax.fori_loop(0, RW, row, 0)
    pltpu.sync_copy(coef_t, coef_hbm.at[wid])


_sc_coef = pl.kernel(
    _sc_coef_body,
    out_type=jax.ShapeDtypeStruct((NW, RW, 128), jnp.float32),
    mesh=_MESH,
    compiler_params=_SC_PARAMS,
    scratch_types=[
        pltpu.VMEM((N,), jnp.float32),
        pltpu.VMEM((N,), jnp.float32),
        pltpu.VMEM((N,), jnp.float32),
        pltpu.VMEM((N,), jnp.float32),
        pltpu.VMEM((RW, 128), jnp.int32),
        pltpu.VMEM((RW, 128), jnp.int32),
        pltpu.VMEM((RW, 128), jnp.float32),
    ],
)


def _sc_hop_body(aa_hbm, ab_hbm, zvec_hbm, coef_hbm, edge_hbm, out_hbm,
                 buf_a, buf_b, coef_t, src_t, dst_t, val_t, acc):
    cid, sid, wid = _worker_id()
    _stage_sum2(aa_hbm, ab_hbm, buf_a, buf_b)
    pltpu.sync_copy(coef_hbm.at[wid], coef_t)
    pltpu.sync_copy(edge_hbm.at[0, wid], src_t)
    pltpu.sync_copy(edge_hbm.at[1, wid], dst_t)

    @pl.when((sid == 0) & (cid == 0))
    def _():
        pltpu.sync_copy(buf_a, acc)

    @pl.when((sid == 0) & (cid == 1))
    def _():
        pltpu.sync_copy(zvec_hbm, acc)

    def row(j, _):
        for k in range(8):
            s = pl.ds(k * 16, 16)
            di = dst_t[j, s]
            g = plsc.load_gather(buf_a, [di])
            val_t[j, s] = g * coef_t[j, s] * _valid(j, k).astype(jnp.float32)
        return 0

    lax.fori_loop(0, RW, row, 0)
    plsc.subcore_barrier()

    def srow(j, _):
        pltpu.sync_copy(val_t.at[j], acc.at[src_t.at[j]], add=True)
        return 0

    lax.fori_loop(0, RW, srow, 0)
    plsc.subcore_barrier()

    @pl.when(sid == 0)
    def _():
        pltpu.sync_copy(acc, out_hbm.at[cid])


_sc_hop = pl.kernel(
    _sc_hop_body,
    out_type=jax.ShapeDtypeStruct((NC, N), jnp.float32),
    mesh=_MESH,
    compiler_params=_SC_PARAMS,
    scratch_types=[
        pltpu.VMEM((N,), jnp.float32),
        pltpu.VMEM((N,), jnp.float32),
        pltpu.VMEM((RW, 128), jnp.float32),
        pltpu.VMEM((RW, 128), jnp.int32),
        pltpu.VMEM((RW, 128), jnp.int32),
        pltpu.VMEM((RW, 128), jnp.float32),
        pltpu.VMEM_SHARED((N,), jnp.float32),
    ],
)


def _pre_kernel(yh_ref, ys_ref, vin_ref, tout_ref, nsq_ref, nrm_ref):
    bq = yh_ref.shape[0]
    tout_ref[...] = jnp.sum((ys_ref[...] - yh_ref[...]), axis=0) / (
        bq * np.sqrt(D)
    )
    nsq = jnp.sum(vin_ref[...] * vin_ref[...], axis=1)
    nsq_ref[...] = nsq
    nrm_ref[...] = jnp.sqrt(nsq)


def _dense_kernel(vout_ref, vw_ref, good_ref, nsq_ref, aa_ref, ab_ref,
                  tout_ref, out_ref):
    theta = THETA0 * (nsq_ref[...] + 1e-9)
    delta_g = good_ref[...] - theta
    f_prime = 1.0 - jnp.tanh(vw_ref[...]) ** 2
    a = aa_ref[...] + ab_ref[...]
    t_rows = a[:, None] * tout_ref[...][None, :]
    dw = LAM_C * vout_ref[...] * f_prime * delta_g[:, None] + LAM_R * (
        t_rows * f_prime
    )
    g = jnp.sqrt(jnp.sum(dw * dw, axis=1, keepdims=True))
    out_ref[...] = jnp.where(g > 5.0, dw * (5.0 / (g + 1e-12)), dw)


def kernel(Y_hat, Y_star, V_in, V_out, V_weighted, goodness, spec_norm, edge_index):
    n = V_in.shape[0]
    tout, nsq, nrm = pl.pallas_call(
        _pre_kernel,
        out_shape=[
            jax.ShapeDtypeStruct((D,), jnp.float32),
            jax.ShapeDtypeStruct((n,), jnp.float32),
            jax.ShapeDtypeStruct((n,), jnp.float32),
        ],
    )(Y_hat, Y_star, V_in)

    pad = NW * RW * 128 - E
    pad_src = jnp.zeros((pad,), jnp.int32)
    pad_dst = (jnp.arange(pad, dtype=jnp.int32) * 37) % n
    src_p = jnp.concatenate([edge_index[0], pad_src]).reshape(NW, RW, 128)
    dst_p = jnp.concatenate([edge_index[1], pad_dst]).reshape(NW, RW, 128)
    edges = jnp.stack([src_p, dst_p])

    zvec = jnp.zeros((n,), jnp.float32)
    evec = jnp.zeros((n,), jnp.float32).at[n - 1].set(1.0)

    tot2 = _sc_total(nrm, edges, zvec)
    coef = _sc_coef(nrm, spec_norm, tot2, edges)
    a2 = _sc_hop(evec, zvec, zvec, coef, edges)
    for _ in range(HOPS - 1):
        a2 = _sc_hop(a2[0], a2[1], zvec, coef, edges)

    dw = pl.pallas_call(
        _dense_kernel,
        out_shape=jax.ShapeDtypeStruct((n, D), jnp.float32),
    )(V_out, V_weighted, goodness, nsq, a2[0], a2[1], tout)
    return dw


# speedup vs baseline: 81.7254x; 1.4338x over previous
import functools

import jax
import jax.numpy as jnp
import numpy as np
from jax import lax
from jax.experimental import pallas as pl
from jax.experimental.pallas import tpu as pltpu
from jax.experimental.pallas import tpu_sc as plsc


N = 10000
E = 320000
D = 128
LAM_C = 0.65
LAM_R = 0.35
THETA0 = 0.5
SPEC_MIN = 0.3
SPEC_MAX = 4.0
HOPS = 3

NC = 2
NS = 16
NW = NC * NS
EW = E // NW
RW = 80
RC = 8
NG = N // 16

_MESH = plsc.VectorSubcoreMesh(
    core_axis_name="c", subcore_axis_name="s", num_cores=NC, num_subcores=NS
)
_SC_PARAMS = pltpu.CompilerParams(needs_layout_passes=False)


def _worker_id():
    cid = lax.axis_index("c")
    sid = lax.axis_index("s")
    return cid, sid, cid * NS + sid


def _wait_all(descs):
    for d in descs:
        d.wait()


def _stage_sum2_local(buf_a, buf_b, bias=0.0):
    def body(i, _):
        s = pl.ds(i * 16, 16)
        buf_a[s] = buf_a[s] + buf_b[s] + bias
        return 0

    lax.fori_loop(0, NG, body, 0)


def _valid(j, k):
    return ((j < EW // 128) | ((j == EW // 128) & (k < (EW % 128) // 16))).astype(
        jnp.float32
    )


def _scatter_rows(compute_row, idx_t, val_t, acc, sem):
    def chunk(c, _):
        descs = []
        for i in range(RC):
            j = c * RC + i
            compute_row(j)
            descs.append(
                pltpu.async_copy(val_t.at[j], acc.at[idx_t.at[j]], sem, add=True)
            )
        _wait_all(descs)
        return 0

    lax.fori_loop(0, RW // RC, chunk, 0)


def _sc_total_body(nrm_hbm, edge_hbm, zvec_hbm, out_hbm,
                   nrm_loc, src_t, dst_t, val_t, acc, sem):
    cid, sid, wid = _worker_id()
    _wait_all([
        pltpu.async_copy(nrm_hbm, nrm_loc, sem),
        pltpu.async_copy(edge_hbm.at[0, wid], src_t, sem),
        pltpu.async_copy(edge_hbm.at[1, wid], dst_t, sem),
    ])

    @pl.when(sid == 0)
    def _():
        pltpu.sync_copy(zvec_hbm, acc)

    plsc.subcore_barrier()

    def compute_row(j):
        for k in range(8):
            s = pl.ds(k * 16, 16)
            v = plsc.load_gather(nrm_loc, [src_t[j, s]])
            val_t[j, s] = v * _valid(j, k)

    _scatter_rows(compute_row, dst_t, val_t, acc, sem)
    plsc.subcore_barrier()

    @pl.when(sid == 0)
    def _():
        pltpu.sync_copy(acc, out_hbm.at[cid])


_sc_total = pl.kernel(
    _sc_total_body,
    out_type=jax.ShapeDtypeStruct((NC, N), jnp.float32),
    mesh=_MESH,
    compiler_params=_SC_PARAMS,
    scratch_types=[
        pltpu.VMEM((N,), jnp.float32),
        pltpu.VMEM((RW, 128), jnp.int32),
        pltpu.VMEM((RW, 128), jnp.int32),
        pltpu.VMEM((RW, 128), jnp.float32),
        pltpu.VMEM_SHARED((N,), jnp.float32),
        pltpu.SemaphoreType.DMA,
    ],
)


def _sc_coef_hop1_body(nrm_hbm, spec_hbm, tot2_hbm, edge_hbm, evec_hbm, zvec_hbm,
                       coef_hbm, aout_hbm,
                       nrm_loc, t_loc, t_b, spec_loc, src_t, dst_t, coef_t,
                       val_t, acc, sem):
    cid, sid, wid = _worker_id()
    _wait_all([
        pltpu.async_copy(nrm_hbm, nrm_loc, sem),
        pltpu.async_copy(spec_hbm, spec_loc, sem),
        pltpu.async_copy(tot2_hbm.at[0], t_loc, sem),
        pltpu.async_copy(tot2_hbm.at[1], t_b, sem),
        pltpu.async_copy(edge_hbm.at[0, wid], src_t, sem),
        pltpu.async_copy(edge_hbm.at[1, wid], dst_t, sem),
    ])
    _stage_sum2_local(t_loc, t_b, bias=1e-9)

    @pl.when((sid == 0) & (cid == 0))
    def _():
        pltpu.sync_copy(evec_hbm, acc)

    @pl.when((sid == 0) & (cid == 1))
    def _():
        pltpu.sync_copy(zvec_hbm, acc)

    plsc.subcore_barrier()
    out_id = jnp.int32(N - 1)

    def compute_row(j):
        for k in range(8):
            s = pl.ds(k * 16, 16)
            si = src_t[j, s]
            di = dst_t[j, s]
            v = plsc.load_gather(nrm_loc, [si])
            t = plsc.load_gather(t_loc, [di])
            sp = plsc.load_gather(spec_loc, [di])
            sp = jnp.minimum(jnp.maximum(sp, SPEC_MIN), SPEC_MAX)
            c = v / t * sp * _valid(j, k)
            coef_t[j, s] = c
            val_t[j, s] = c * (di == out_id).astype(jnp.float32)

    _scatter_rows(compute_row, src_t, val_t, acc, sem)
    pltpu.sync_copy(coef_t, coef_hbm.at[wid])
    plsc.subcore_barrier()

    @pl.when(sid == 0)
    def _():
        pltpu.sync_copy(acc, aout_hbm.at[cid])


_sc_coef_hop1 = pl.kernel(
    _sc_coef_hop1_body,
    out_type=[
        jax.ShapeDtypeStruct((NW, RW, 128), jnp.float32),
        jax.ShapeDtypeStruct((NC, N), jnp.float32),
    ],
    mesh=_MESH,
    compiler_params=_SC_PARAMS,
    scratch_types=[
        pltpu.VMEM((N,), jnp.float32),
        pltpu.VMEM((N,), jnp.float32),
        pltpu.VMEM((N,), jnp.float32),
        pltpu.VMEM((N,), jnp.float32),
        pltpu.VMEM((RW, 128), jnp.int32),
        pltpu.VMEM((RW, 128), jnp.int32),
        pltpu.VMEM((RW, 128), jnp.float32),
        pltpu.VMEM((RW, 128), jnp.float32),
        pltpu.VMEM_SHARED((N,), jnp.float32),
        pltpu.SemaphoreType.DMA,
    ],
)


def _sc_hop_body(a2_hbm, zvec_hbm, coef_hbm, edge_hbm, out_hbm,
                 buf_a, buf_b, coef_t, src_t, dst_t, val_t, acc, sem):
    cid, sid, wid = _worker_id()
    _wait_all([
        pltpu.async_copy(a2_hbm.at[0], buf_a, sem),
        pltpu.async_copy(a2_hbm.at[1], buf_b, sem),
        pltpu.async_copy(coef_hbm.at[wid], coef_t, sem),
        pltpu.async_copy(edge_hbm.at[0, wid], src_t, sem),
        pltpu.async_copy(edge_hbm.at[1, wid], dst_t, sem),
    ])
    _stage_sum2_local(buf_a, buf_b)

    @pl.when((sid == 0) & (cid == 0))
    def _():
        pltpu.sync_copy(buf_a, acc)

    @pl.when((sid == 0) & (cid == 1))
    def _():
        pltpu.sync_copy(zvec_hbm, acc)

    plsc.subcore_barrier()

    def compute_row(j):
        for k in range(8):
            s = pl.ds(k * 16, 16)
            g = plsc.load_gather(buf_a, [dst_t[j, s]])
            val_t[j, s] = g * coef_t[j, s]

    _scatter_rows(compute_row, src_t, val_t, acc, sem)
    plsc.subcore_barrier()

    @pl.when(sid == 0)
    def _():
        pltpu.sync_copy(acc, out_hbm.at[cid])


_sc_hop = pl.kernel(
    _sc_hop_body,
    out_type=jax.ShapeDtypeStruct((NC, N), jnp.float32),
    mesh=_MESH,
    compiler_params=_SC_PARAMS,
    scratch_types=[
        pltpu.VMEM((N,), jnp.float32),
        pltpu.VMEM((N,), jnp.float32),
        pltpu.VMEM((RW, 128), jnp.float32),
        pltpu.VMEM((RW, 128), jnp.int32),
        pltpu.VMEM((RW, 128), jnp.int32),
        pltpu.VMEM((RW, 128), jnp.float32),
        pltpu.VMEM_SHARED((N,), jnp.float32),
        pltpu.SemaphoreType.DMA,
    ],
)


def _pre_kernel(yh_ref, ys_ref, vin_ref, tout_ref, nsq_ref, nrm_ref):
    bq = yh_ref.shape[0]
    tout_ref[...] = jnp.sum((ys_ref[...] - yh_ref[...]), axis=0) / (
        bq * np.sqrt(D)
    )
    nsq = jnp.sum(vin_ref[...] * vin_ref[...], axis=1)
    nsq_ref[...] = nsq
    nrm_ref[...] = jnp.sqrt(nsq)


def _dense_kernel(vout_ref, vw_ref, good_ref, nsq_ref, a2_ref,
                  tout_ref, out_ref):
    theta = THETA0 * (nsq_ref[...] + 1e-9)
    delta_g = good_ref[...] - theta
    f_prime = 1.0 - jnp.tanh(vw_ref[...]) ** 2
    a = a2_ref[0, :] + a2_ref[1, :]
    t_rows = a[:, None] * tout_ref[...][None, :]
    dw = LAM_C * vout_ref[...] * f_prime * delta_g[:, None] + LAM_R * (
        t_rows * f_prime
    )
    g = jnp.sqrt(jnp.sum(dw * dw, axis=1, keepdims=True))
    out_ref[...] = jnp.where(g > 5.0, dw * (5.0 / (g + 1e-12)), dw)


def kernel(Y_hat, Y_star, V_in, V_out, V_weighted, goodness, spec_norm, edge_index):
    n = V_in.shape[0]
    tout, nsq, nrm = pl.pallas_call(
        _pre_kernel,
        out_shape=[
            jax.ShapeDtypeStruct((D,), jnp.float32),
            jax.ShapeDtypeStruct((n,), jnp.float32),
            jax.ShapeDtypeStruct((n,), jnp.float32),
        ],
    )(Y_hat, Y_star, V_in)

    pad = NW * RW * 128 - E
    pad_src = jnp.zeros((pad,), jnp.int32)
    pad_dst = (jnp.arange(pad, dtype=jnp.int32) * 37) % n
    src_p = jnp.concatenate([edge_index[0], pad_src]).reshape(NW, RW, 128)
    dst_p = jnp.concatenate([edge_index[1], pad_dst]).reshape(NW, RW, 128)
    edges = jnp.stack([src_p, dst_p])

    zvec = jnp.zeros((n,), jnp.float32)
    evec = jnp.zeros((n,), jnp.float32).at[n - 1].set(1.0)

    tot2 = _sc_total(nrm, edges, zvec)
    coef, a2 = _sc_coef_hop1(nrm, spec_norm, tot2, edges, evec, zvec)
    for _ in range(HOPS - 1):
        a2 = _sc_hop(a2, zvec, coef, edges)

    dw = pl.pallas_call(
        _dense_kernel,
        out_shape=jax.ShapeDtypeStruct((n, D), jnp.float32),
    )(V_out, V_weighted, goodness, nsq, a2, tout)
    return dw


# speedup vs baseline: 85.9265x; 1.0514x over previous
import functools

import jax
import jax.numpy as jnp
import numpy as np
from jax import lax
from jax.experimental import pallas as pl
from jax.experimental.pallas import tpu as pltpu
from jax.experimental.pallas import tpu_sc as plsc


N = 10000
E = 320000
D = 128
LAM_C = 0.65
LAM_R = 0.35
THETA0 = 0.5
SPEC_MIN = 0.3
SPEC_MAX = 4.0
HOPS = 3

NC = 2
NS = 16
NW = NC * NS
EW = E // NW
RW = 80
RC = 8
NG = N // 16

_MESH = plsc.VectorSubcoreMesh(
    core_axis_name="c", subcore_axis_name="s", num_cores=NC, num_subcores=NS
)
_SC_PARAMS = pltpu.CompilerParams(needs_layout_passes=False)


def _worker_id():
    cid = lax.axis_index("c")
    sid = lax.axis_index("s")
    return cid, sid, cid * NS + sid


def _wait_all(descs):
    for d in descs:
        d.wait()


def _valid(j, k):
    return ((j < EW // 128) | ((j == EW // 128) & (k < (EW % 128) // 16))).astype(
        jnp.float32
    )


def _scatter_rows(compute_row, idx_t, val_t, acc, sem):
    def chunk(c, _):
        descs = []
        for i in range(RC):
            j = c * RC + i
            compute_row(j)
            descs.append(
                pltpu.async_copy(val_t.at[j], acc.at[idx_t.at[j]], sem, add=True)
            )
        _wait_all(descs)
        return 0

    lax.fori_loop(0, RW // RC, chunk, 0)


def _sc_total_body(nrm_hbm, edge_hbm, zvec_hbm, out_hbm,
                   nrm_loc, src_t, dst_t, val_t, acc, sem):
    cid, sid, wid = _worker_id()
    _wait_all([
        pltpu.async_copy(nrm_hbm, nrm_loc, sem),
        pltpu.async_copy(edge_hbm.at[0, wid], src_t, sem),
        pltpu.async_copy(edge_hbm.at[1, wid], dst_t, sem),
    ])

    @pl.when(sid == 0)
    def _():
        pltpu.sync_copy(zvec_hbm, acc)

    plsc.subcore_barrier()

    def compute_row(j):
        for k in range(8):
            s = pl.ds(k * 16, 16)
            v = plsc.load_gather(nrm_loc, [src_t[j, s]])
            val_t[j, s] = v * _valid(j, k)

    _scatter_rows(compute_row, dst_t, val_t, acc, sem)
    plsc.subcore_barrier()

    @pl.when(sid == 0)
    def _():
        pltpu.sync_copy(acc, out_hbm.at[cid])


_sc_total = pl.kernel(
    _sc_total_body,
    out_type=jax.ShapeDtypeStruct((NC, N), jnp.float32),
    mesh=_MESH,
    compiler_params=_SC_PARAMS,
    scratch_types=[
        pltpu.VMEM((N,), jnp.float32),
        pltpu.VMEM((RW, 128), jnp.int32),
        pltpu.VMEM((RW, 128), jnp.int32),
        pltpu.VMEM((RW, 128), jnp.float32),
        pltpu.VMEM_SHARED((N,), jnp.float32),
        pltpu.SemaphoreType.DMA,
    ],
)


def _sc_coef_hop1_body(nrm_hbm, spec_hbm, tot2_hbm, edge_hbm, evec_hbm, zvec_hbm,
                       coef_hbm, aout_hbm,
                       nrm_loc, t_loc, t_b, spec_loc, src_t, dst_t, coef_t,
                       val_t, acc, sem):
    cid, sid, wid = _worker_id()
    _wait_all([
        pltpu.async_copy(nrm_hbm, nrm_loc, sem),
        pltpu.async_copy(spec_hbm, spec_loc, sem),
        pltpu.async_copy(tot2_hbm.at[0], t_loc, sem),
        pltpu.async_copy(tot2_hbm.at[1], t_b, sem),
        pltpu.async_copy(edge_hbm.at[0, wid], src_t, sem),
        pltpu.async_copy(edge_hbm.at[1, wid], dst_t, sem),
    ])

    @pl.when((sid == 0) & (cid == 0))
    def _():
        pltpu.sync_copy(evec_hbm, acc)

    @pl.when((sid == 0) & (cid == 1))
    def _():
        pltpu.sync_copy(zvec_hbm, acc)

    plsc.subcore_barrier()
    out_id = jnp.int32(N - 1)

    def compute_row(j):
        for k in range(8):
            s = pl.ds(k * 16, 16)
            si = src_t[j, s]
            di = dst_t[j, s]
            v = plsc.load_gather(nrm_loc, [si])
            t = plsc.load_gather(t_loc, [di]) + plsc.load_gather(t_b, [di]) + 1e-9
            sp = plsc.load_gather(spec_loc, [di])
            sp = jnp.minimum(jnp.maximum(sp, SPEC_MIN), SPEC_MAX)
            c = v / t * sp * _valid(j, k)
            coef_t[j, s] = c
            val_t[j, s] = c * (di == out_id).astype(jnp.float32)

    _scatter_rows(compute_row, src_t, val_t, acc, sem)
    pltpu.sync_copy(coef_t, coef_hbm.at[wid])
    plsc.subcore_barrier()

    @pl.when(sid == 0)
    def _():
        pltpu.sync_copy(acc, aout_hbm.at[cid])


_sc_coef_hop1 = pl.kernel(
    _sc_coef_hop1_body,
    out_type=[
        jax.ShapeDtypeStruct((NW, RW, 128), jnp.float32),
        jax.ShapeDtypeStruct((NC, N), jnp.float32),
    ],
    mesh=_MESH,
    compiler_params=_SC_PARAMS,
    scratch_types=[
        pltpu.VMEM((N,), jnp.float32),
        pltpu.VMEM((N,), jnp.float32),
        pltpu.VMEM((N,), jnp.float32),
        pltpu.VMEM((N,), jnp.float32),
        pltpu.VMEM((RW, 128), jnp.int32),
        pltpu.VMEM((RW, 128), jnp.int32),
        pltpu.VMEM((RW, 128), jnp.float32),
        pltpu.VMEM((RW, 128), jnp.float32),
        pltpu.VMEM_SHARED((N,), jnp.float32),
        pltpu.SemaphoreType.DMA,
    ],
)


def _sc_hop_body(a2_hbm, coef_hbm, edge_hbm, out_hbm,
                 buf_a, buf_b, coef_t, src_t, dst_t, val_t, acc, sem):
    cid, sid, wid = _worker_id()
    _wait_all([
        pltpu.async_copy(a2_hbm.at[0], buf_a, sem),
        pltpu.async_copy(a2_hbm.at[1], buf_b, sem),
        pltpu.async_copy(coef_hbm.at[wid], coef_t, sem),
        pltpu.async_copy(edge_hbm.at[0, wid], src_t, sem),
        pltpu.async_copy(edge_hbm.at[1, wid], dst_t, sem),
    ])

    @pl.when(sid == 0)
    def _():
        pltpu.sync_copy(a2_hbm.at[cid], acc)

    plsc.subcore_barrier()

    def compute_row(j):
        for k in range(8):
            s = pl.ds(k * 16, 16)
            di = dst_t[j, s]
            g = plsc.load_gather(buf_a, [di]) + plsc.load_gather(buf_b, [di])
            val_t[j, s] = g * coef_t[j, s]

    _scatter_rows(compute_row, src_t, val_t, acc, sem)
    plsc.subcore_barrier()

    @pl.when(sid == 0)
    def _():
        pltpu.sync_copy(acc, out_hbm.at[cid])


_sc_hop = pl.kernel(
    _sc_hop_body,
    out_type=jax.ShapeDtypeStruct((NC, N), jnp.float32),
    mesh=_MESH,
    compiler_params=_SC_PARAMS,
    scratch_types=[
        pltpu.VMEM((N,), jnp.float32),
        pltpu.VMEM((N,), jnp.float32),
        pltpu.VMEM((RW, 128), jnp.float32),
        pltpu.VMEM((RW, 128), jnp.int32),
        pltpu.VMEM((RW, 128), jnp.int32),
        pltpu.VMEM((RW, 128), jnp.float32),
        pltpu.VMEM_SHARED((N,), jnp.float32),
        pltpu.SemaphoreType.DMA,
    ],
)


def _pre_kernel(yh_ref, ys_ref, vin_ref, tout_ref, nsq_ref, nrm_ref):
    bq = yh_ref.shape[0]
    tout_ref[...] = jnp.sum((ys_ref[...] - yh_ref[...]), axis=0) / (
        bq * np.sqrt(D)
    )
    nsq = jnp.sum(vin_ref[...] * vin_ref[...], axis=1)
    nsq_ref[...] = nsq
    nrm_ref[...] = jnp.sqrt(nsq)


def _dense_kernel(vout_ref, vw_ref, good_ref, nsq_ref, a2_ref,
                  tout_ref, out_ref):
    theta = THETA0 * (nsq_ref[...] + 1e-9)
    delta_g = good_ref[...] - theta
    f_prime = 1.0 - jnp.tanh(vw_ref[...]) ** 2
    a = a2_ref[0, :] + a2_ref[1, :]
    t_rows = a[:, None] * tout_ref[...][None, :]
    dw = LAM_C * vout_ref[...] * f_prime * delta_g[:, None] + LAM_R * (
        t_rows * f_prime
    )
    g = jnp.sqrt(jnp.sum(dw * dw, axis=1, keepdims=True))
    out_ref[...] = jnp.where(g > 5.0, dw * (5.0 / (g + 1e-12)), dw)


def kernel(Y_hat, Y_star, V_in, V_out, V_weighted, goodness, spec_norm, edge_index):
    n = V_in.shape[0]
    tout, nsq, nrm = pl.pallas_call(
        _pre_kernel,
        out_shape=[
            jax.ShapeDtypeStruct((D,), jnp.float32),
            jax.ShapeDtypeStruct((n,), jnp.float32),
            jax.ShapeDtypeStruct((n,), jnp.float32),
        ],
    )(Y_hat, Y_star, V_in)

    pad = NW * RW * 128 - E
    pad_src = jnp.zeros((pad,), jnp.int32)
    pad_dst = (jnp.arange(pad, dtype=jnp.int32) * 37) % n
    src_p = jnp.concatenate([edge_index[0], pad_src]).reshape(NW, RW, 128)
    dst_p = jnp.concatenate([edge_index[1], pad_dst]).reshape(NW, RW, 128)
    edges = jnp.stack([src_p, dst_p])

    zvec = jnp.zeros((n,), jnp.float32)
    evec = jnp.zeros((n,), jnp.float32).at[n - 1].set(1.0)

    tot2 = _sc_total(nrm, edges, zvec)
    coef, a2 = _sc_coef_hop1(nrm, spec_norm, tot2, edges, evec, zvec)
    for _ in range(HOPS - 1):
        a2 = _sc_hop(a2, coef, edges)

    dw = pl.pallas_call(
        _dense_kernel,
        out_shape=jax.ShapeDtypeStruct((n, D), jnp.float32),
    )(V_out, V_weighted, goodness, nsq, a2, tout)
    return dw


# speedup vs baseline: 87.6041x; 1.0195x over previous
import functools

import jax
import jax.numpy as jnp
import numpy as np
from jax import lax
from jax.experimental import pallas as pl
from jax.experimental.pallas import tpu as pltpu
from jax.experimental.pallas import tpu_sc as plsc


N = 10000
E = 320000
D = 128
LAM_C = 0.65
LAM_R = 0.35
THETA0 = 0.5
SPEC_MIN = 0.3
SPEC_MAX = 4.0
HOPS = 3

NC = 2
NS = 16
NW = NC * NS
EW = E // NW
RW = 80
RC = 8
NG = N // 16

_MESH = plsc.VectorSubcoreMesh(
    core_axis_name="c", subcore_axis_name="s", num_cores=NC, num_subcores=NS
)
_SC_PARAMS = pltpu.CompilerParams(needs_layout_passes=False)


def _worker_id():
    cid = lax.axis_index("c")
    sid = lax.axis_index("s")
    return cid, sid, cid * NS + sid


def _wait_all(descs):
    for d in descs:
        d.wait()


def _valid(j, k):
    return ((j < EW // 128) | ((j == EW // 128) & (k < (EW % 128) // 16))).astype(
        jnp.float32
    )


def _scatter_rows(compute_row, idx_t, val_t, acc, sem):
    def chunk(c, _):
        descs = []
        for i in range(RC):
            j = c * RC + i
            compute_row(j)
            descs.append(
                pltpu.async_copy(val_t.at[j], acc.at[idx_t.at[j]], sem, add=True)
            )
        _wait_all(descs)
        return 0

    lax.fori_loop(0, RW // RC, chunk, 0)


def _sc_total_body(nrm_hbm, edge_hbm, zvec_hbm, out_hbm,
                   nrm_loc, src_t, dst_t, val_t, acc, sem):
    cid, sid, wid = _worker_id()
    _wait_all([
        pltpu.async_copy(nrm_hbm, nrm_loc, sem),
        pltpu.async_copy(edge_hbm.at[0, wid], src_t, sem),
        pltpu.async_copy(edge_hbm.at[1, wid], dst_t, sem),
    ])

    @pl.when(sid == 0)
    def _():
        pltpu.sync_copy(zvec_hbm, acc)

    plsc.subcore_barrier()

    def compute_row(j):
        for k in range(8):
            s = pl.ds(k * 16, 16)
            v = plsc.load_gather(nrm_loc, [src_t[j, s]])
            val_t[j, s] = v * _valid(j, k)

    _scatter_rows(compute_row, dst_t, val_t, acc, sem)
    plsc.subcore_barrier()

    @pl.when(sid == 0)
    def _():
        pltpu.sync_copy(acc, out_hbm.at[cid])


_sc_total = pl.kernel(
    _sc_total_body,
    out_type=jax.ShapeDtypeStruct((NC, N), jnp.float32),
    mesh=_MESH,
    compiler_params=_SC_PARAMS,
    scratch_types=[
        pltpu.VMEM((N,), jnp.float32),
        pltpu.VMEM((RW, 128), jnp.int32),
        pltpu.VMEM((RW, 128), jnp.int32),
        pltpu.VMEM((RW, 128), jnp.float32),
        pltpu.VMEM_SHARED((N,), jnp.float32),
        pltpu.SemaphoreType.DMA,
    ],
)


def _sc_coef_hop1_body(nrm_hbm, spec_hbm, tot2_hbm, edge_hbm, evec_hbm, zvec_hbm,
                       coef_hbm, aout_hbm,
                       nrm_loc, t_loc, t_b, spec_loc, src_t, dst_t, coef_t,
                       val_t, acc, sem):
    cid, sid, wid = _worker_id()
    _wait_all([
        pltpu.async_copy(nrm_hbm, nrm_loc, sem),
        pltpu.async_copy(spec_hbm, spec_loc, sem),
        pltpu.async_copy(tot2_hbm.at[0], t_loc, sem),
        pltpu.async_copy(tot2_hbm.at[1], t_b, sem),
        pltpu.async_copy(edge_hbm.at[0, wid], src_t, sem),
        pltpu.async_copy(edge_hbm.at[1, wid], dst_t, sem),
    ])

    @pl.when((sid == 0) & (cid == 0))
    def _():
        pltpu.sync_copy(evec_hbm, acc)

    @pl.when((sid == 0) & (cid == 1))
    def _():
        pltpu.sync_copy(zvec_hbm, acc)

    plsc.subcore_barrier()
    out_id = jnp.int32(N - 1)

    def row_body(j, _):
        hits = jnp.zeros((16,), jnp.int32)
        for k in range(8):
            s = pl.ds(k * 16, 16)
            si = src_t[j, s]
            di = dst_t[j, s]
            v = plsc.load_gather(nrm_loc, [si])
            t = plsc.load_gather(t_loc, [di]) + plsc.load_gather(t_b, [di]) + 1e-9
            sp = plsc.load_gather(spec_loc, [di])
            sp = jnp.minimum(jnp.maximum(sp, SPEC_MIN), SPEC_MAX)
            c = v / t * sp * _valid(j, k)
            coef_t[j, s] = c
            hit = (di == out_id).astype(jnp.int32)
            hits = hits + hit
            val_t[j, s] = c * hit.astype(jnp.float32)

        @pl.when(jnp.sum(hits) > 0)
        def _():
            pltpu.sync_copy(val_t.at[j], acc.at[src_t.at[j]], add=True)

        return 0

    lax.fori_loop(0, RW, row_body, 0)
    pltpu.sync_copy(coef_t, coef_hbm.at[wid])
    plsc.subcore_barrier()

    @pl.when(sid == 0)
    def _():
        pltpu.sync_copy(acc, aout_hbm.at[cid])


_sc_coef_hop1 = pl.kernel(
    _sc_coef_hop1_body,
    out_type=[
        jax.ShapeDtypeStruct((NW, RW, 128), jnp.float32),
        jax.ShapeDtypeStruct((NC, N), jnp.float32),
    ],
    mesh=_MESH,
    compiler_params=_SC_PARAMS,
    scratch_types=[
        pltpu.VMEM((N,), jnp.float32),
        pltpu.VMEM((N,), jnp.float32),
        pltpu.VMEM((N,), jnp.float32),
        pltpu.VMEM((N,), jnp.float32),
        pltpu.VMEM((RW, 128), jnp.int32),
        pltpu.VMEM((RW, 128), jnp.int32),
        pltpu.VMEM((RW, 128), jnp.float32),
        pltpu.VMEM((RW, 128), jnp.float32),
        pltpu.VMEM_SHARED((N,), jnp.float32),
        pltpu.SemaphoreType.DMA,
    ],
)


def _sc_hop_body(a2_hbm, coef_hbm, edge_hbm, out_hbm,
                 buf_a, buf_b, coef_t, src_t, dst_t, val_t, acc, sem):
    cid, sid, wid = _worker_id()
    _wait_all([
        pltpu.async_copy(a2_hbm.at[0], buf_a, sem),
        pltpu.async_copy(a2_hbm.at[1], buf_b, sem),
        pltpu.async_copy(coef_hbm.at[wid], coef_t, sem),
        pltpu.async_copy(edge_hbm.at[0, wid], src_t, sem),
        pltpu.async_copy(edge_hbm.at[1, wid], dst_t, sem),
    ])

    @pl.when(sid == 0)
    def _():
        pltpu.sync_copy(a2_hbm.at[cid], acc)

    plsc.subcore_barrier()

    def compute_row(j):
        for k in range(8):
            s = pl.ds(k * 16, 16)
            di = dst_t[j, s]
            g = plsc.load_gather(buf_a, [di]) + plsc.load_gather(buf_b, [di])
            val_t[j, s] = g * coef_t[j, s]

    _scatter_rows(compute_row, src_t, val_t, acc, sem)
    plsc.subcore_barrier()

    @pl.when(sid == 0)
    def _():
        pltpu.sync_copy(acc, out_hbm.at[cid])


_sc_hop = pl.kernel(
    _sc_hop_body,
    out_type=jax.ShapeDtypeStruct((NC, N), jnp.float32),
    mesh=_MESH,
    compiler_params=_SC_PARAMS,
    scratch_types=[
        pltpu.VMEM((N,), jnp.float32),
        pltpu.VMEM((N,), jnp.float32),
        pltpu.VMEM((RW, 128), jnp.float32),
        pltpu.VMEM((RW, 128), jnp.int32),
        pltpu.VMEM((RW, 128), jnp.int32),
        pltpu.VMEM((RW, 128), jnp.float32),
        pltpu.VMEM_SHARED((N,), jnp.float32),
        pltpu.SemaphoreType.DMA,
    ],
)


def _pre_kernel(yh_ref, ys_ref, vin_ref, tout_ref, nsq_ref, nrm_ref):
    bq = yh_ref.shape[0]
    tout_ref[...] = jnp.sum((ys_ref[...] - yh_ref[...]), axis=0) / (
        bq * np.sqrt(D)
    )
    nsq = jnp.sum(vin_ref[...] * vin_ref[...], axis=1)
    nsq_ref[...] = nsq
    nrm_ref[...] = jnp.sqrt(nsq)


def _dense_kernel(vout_ref, vw_ref, good_ref, nsq_ref, a2_ref,
                  tout_ref, out_ref):
    theta = THETA0 * (nsq_ref[...] + 1e-9)
    delta_g = good_ref[...] - theta
    f_prime = 1.0 - jnp.tanh(vw_ref[...]) ** 2
    a = a2_ref[0, :] + a2_ref[1, :]
    t_rows = a[:, None] * tout_ref[...][None, :]
    dw = LAM_C * vout_ref[...] * f_prime * delta_g[:, None] + LAM_R * (
        t_rows * f_prime
    )
    g = jnp.sqrt(jnp.sum(dw * dw, axis=1, keepdims=True))
    out_ref[...] = jnp.where(g > 5.0, dw * (5.0 / (g + 1e-12)), dw)


def kernel(Y_hat, Y_star, V_in, V_out, V_weighted, goodness, spec_norm, edge_index):
    n = V_in.shape[0]
    tout, nsq, nrm = pl.pallas_call(
        _pre_kernel,
        out_shape=[
            jax.ShapeDtypeStruct((D,), jnp.float32),
            jax.ShapeDtypeStruct((n,), jnp.float32),
            jax.ShapeDtypeStruct((n,), jnp.float32),
        ],
    )(Y_hat, Y_star, V_in)

    pad = NW * RW * 128 - E
    pad_src = jnp.zeros((pad,), jnp.int32)
    pad_dst = (jnp.arange(pad, dtype=jnp.int32) * 37) % n
    src_p = jnp.concatenate([edge_index[0], pad_src]).reshape(NW, RW, 128)
    dst_p = jnp.concatenate([edge_index[1], pad_dst]).reshape(NW, RW, 128)
    edges = jnp.stack([src_p, dst_p])

    zvec = jnp.zeros((n,), jnp.float32)
    evec = jnp.zeros((n,), jnp.float32).at[n - 1].set(1.0)

    tot2 = _sc_total(nrm, edges, zvec)
    coef, a2 = _sc_coef_hop1(nrm, spec_norm, tot2, edges, evec, zvec)
    for _ in range(HOPS - 1):
        a2 = _sc_hop(a2, coef, edges)

    dw = pl.pallas_call(
        _dense_kernel,
        out_shape=jax.ShapeDtypeStruct((n, D), jnp.float32),
    )(V_out, V_weighted, goodness, nsq, a2, tout)
    return dw


# speedup vs baseline: 92.5356x; 1.0563x over previous
import functools

import jax
import jax.numpy as jnp
import numpy as np
from jax import lax
from jax.experimental import pallas as pl
from jax.experimental.pallas import tpu as pltpu
from jax.experimental.pallas import tpu_sc as plsc


N = 10000
E = 320000
D = 128
LAM_C = 0.65
LAM_R = 0.35
THETA0 = 0.5
SPEC_MIN = 0.3
SPEC_MAX = 4.0
HOPS = 3
OUT_ID = N - 1

NC = 2
NS = 16
NW = NC * NS
EW = E // NW
RW = 80
RC = 8
NG = N // 16

_MESH = plsc.VectorSubcoreMesh(
    core_axis_name="c", subcore_axis_name="s", num_cores=NC, num_subcores=NS
)
_SC_PARAMS = pltpu.CompilerParams(needs_layout_passes=False)


def _worker_id():
    cid = lax.axis_index("c")
    sid = lax.axis_index("s")
    return cid, sid, cid * NS + sid


def _wait_all(descs):
    for d in descs:
        d.wait()


def _valid(j, k):
    return ((j < EW // 128) | ((j == EW // 128) & (k < (EW % 128) // 16))).astype(
        jnp.float32
    )


def _scatter_rows(compute_row, idx_t, val_t, acc, sem):
    def chunk(c, _):
        descs = []
        for i in range(RC):
            j = c * RC + i
            compute_row(j)
            descs.append(
                pltpu.async_copy(val_t.at[j], acc.at[idx_t.at[j]], sem, add=True)
            )
        _wait_all(descs)
        return 0

    lax.fori_loop(0, RW // RC, chunk, 0)


def _sc_total_body(nrm_hbm, edge_hbm, zvec_hbm, out_hbm,
                   nrm_loc, src_t, dst_t, val_t, acc, sem):
    cid, sid, wid = _worker_id()
    _wait_all([
        pltpu.async_copy(nrm_hbm, nrm_loc, sem),
        pltpu.async_copy(edge_hbm.at[0, wid], src_t, sem),
        pltpu.async_copy(edge_hbm.at[1, wid], dst_t, sem),
    ])

    @pl.when(sid == 0)
    def _():
        pltpu.sync_copy(zvec_hbm, acc)

    plsc.subcore_barrier()

    def compute_row(j):
        for k in range(8):
            s = pl.ds(k * 16, 16)
            v = plsc.load_gather(nrm_loc, [src_t[j, s]])
            val_t[j, s] = v * _valid(j, k)

    _scatter_rows(compute_row, dst_t, val_t, acc, sem)
    plsc.subcore_barrier()

    @pl.when(sid == 0)
    def _():
        pltpu.sync_copy(acc, out_hbm.at[cid])


_sc_total = pl.kernel(
    _sc_total_body,
    out_type=jax.ShapeDtypeStruct((NC, N), jnp.float32),
    mesh=_MESH,
    compiler_params=_SC_PARAMS,
    scratch_types=[
        pltpu.VMEM((N,), jnp.float32),
        pltpu.VMEM((RW, 128), jnp.int32),
        pltpu.VMEM((RW, 128), jnp.int32),
        pltpu.VMEM((RW, 128), jnp.float32),
        pltpu.VMEM_SHARED((N,), jnp.float32),
        pltpu.SemaphoreType.DMA,
    ],
)


def _sc_hop1_body(nrm_hbm, spec_hbm, tot2_hbm, edge_hbm, zvec_hbm,
                  u_hbm, p_hbm, gout_hbm,
                  spec_loc, ta_loc, tb_loc, nrm_loc, p_loc,
                  s8, ta8, tb8, g16, src_t, dst_t, val_t, acc, sem):
    cid, sid, wid = _worker_id()
    tile00 = (sid == 0) & (cid == 0)
    _wait_all([
        pltpu.async_copy(spec_hbm.at[pl.ds(N - 8, 8)], s8, sem),
        pltpu.async_copy(tot2_hbm.at[0, pl.ds(N - 8, 8)], ta8, sem),
        pltpu.async_copy(tot2_hbm.at[1, pl.ds(N - 8, 8)], tb8, sem),
        pltpu.async_copy(edge_hbm.at[0, wid], src_t, sem),
        pltpu.async_copy(edge_hbm.at[1, wid], dst_t, sem),
    ])

    @pl.when(sid == 0)
    def _():
        pltpu.sync_copy(zvec_hbm, acc)

    i7 = jnp.full((16,), 7, jnp.int32)
    sp = plsc.load_gather(s8, [i7])
    sp = jnp.minimum(jnp.maximum(sp, SPEC_MIN), SPEC_MAX)
    gout = sp / (plsc.load_gather(ta8, [i7]) + plsc.load_gather(tb8, [i7]) + 1e-9)

    @pl.when(tile00)
    def _():
        _wait_all([
            pltpu.async_copy(spec_hbm, spec_loc, sem),
            pltpu.async_copy(tot2_hbm.at[0], ta_loc, sem),
            pltpu.async_copy(tot2_hbm.at[1], tb_loc, sem),
            pltpu.async_copy(nrm_hbm, nrm_loc, sem),
        ])

        def pbody(i, _):
            s = pl.ds(i * 16, 16)
            spv = jnp.minimum(jnp.maximum(spec_loc[s], SPEC_MIN), SPEC_MAX)
            p_loc[s] = spv / (ta_loc[s] + tb_loc[s] + 1e-9) * nrm_loc[s]
            return 0

        lax.fori_loop(0, NG, pbody, 0)
        pltpu.sync_copy(p_loc, p_hbm)
        g16[...] = gout
        pltpu.sync_copy(g16, gout_hbm)

    plsc.subcore_barrier()
    out_id = jnp.int32(OUT_ID)

    def row_body(j, _):
        hits = jnp.zeros((16,), jnp.int32)
        for k in range(8):
            s = pl.ds(k * 16, 16)
            hit = (dst_t[j, s] == out_id).astype(jnp.int32)
            hits = hits + hit
            val_t[j, s] = gout * hit.astype(jnp.float32) * _valid(j, k)

        @pl.when(jnp.sum(hits) > 0)
        def _():
            pltpu.sync_copy(val_t.at[j], acc.at[src_t.at[j]], add=True)

        return 0

    lax.fori_loop(0, RW, row_body, 0)
    plsc.subcore_barrier()

    @pl.when(sid == 0)
    def _():
        pltpu.sync_copy(acc, u_hbm.at[cid])


_sc_hop1 = pl.kernel(
    _sc_hop1_body,
    out_type=[
        jax.ShapeDtypeStruct((NC, N), jnp.float32),
        jax.ShapeDtypeStruct((N,), jnp.float32),
        jax.ShapeDtypeStruct((16,), jnp.float32),
    ],
    mesh=_MESH,
    compiler_params=_SC_PARAMS,
    scratch_types=[
        pltpu.VMEM((N,), jnp.float32),
        pltpu.VMEM((N,), jnp.float32),
        pltpu.VMEM((N,), jnp.float32),
        pltpu.VMEM((N,), jnp.float32),
        pltpu.VMEM((N,), jnp.float32),
        pltpu.VMEM((8,), jnp.float32),
        pltpu.VMEM((8,), jnp.float32),
        pltpu.VMEM((8,), jnp.float32),
        pltpu.VMEM((16,), jnp.float32),
        pltpu.VMEM((RW, 128), jnp.int32),
        pltpu.VMEM((RW, 128), jnp.int32),
        pltpu.VMEM((RW, 128), jnp.float32),
        pltpu.VMEM_SHARED((N,), jnp.float32),
        pltpu.SemaphoreType.DMA,
    ],
)


def _sc_hop_body(u_hbm, p_hbm, gout_hbm, edge_hbm, out_hbm,
                 h_loc, ub_loc, p_loc, g16, src_t, dst_t, val_t, acc, sem):
    cid, sid, wid = _worker_id()
    _wait_all([
        pltpu.async_copy(u_hbm.at[0], h_loc, sem),
        pltpu.async_copy(u_hbm.at[1], ub_loc, sem),
        pltpu.async_copy(p_hbm, p_loc, sem),
        pltpu.async_copy(gout_hbm, g16, sem),
        pltpu.async_copy(edge_hbm.at[0, wid], src_t, sem),
        pltpu.async_copy(edge_hbm.at[1, wid], dst_t, sem),
    ])

    @pl.when(sid == 0)
    def _():
        pltpu.sync_copy(u_hbm.at[cid], acc)

    def hbody(i, _):
        s = pl.ds(i * 16, 16)
        h_loc[s] = p_loc[s] * (h_loc[s] + ub_loc[s])
        return 0

    lax.fori_loop(0, NG, hbody, 0)
    onehot = (lax.iota(jnp.int32, 16) == 15).astype(jnp.float32)
    lastg = pl.ds(16 * (NG - 1), 16)
    h_loc[lastg] = h_loc[lastg] + g16[...] * onehot

    plsc.subcore_barrier()

    def compute_row(j):
        for k in range(8):
            s = pl.ds(k * 16, 16)
            g = plsc.load_gather(h_loc, [dst_t[j, s]])
            val_t[j, s] = g * _valid(j, k)

    _scatter_rows(compute_row, src_t, val_t, acc, sem)
    plsc.subcore_barrier()

    @pl.when(sid == 0)
    def _():
        pltpu.sync_copy(acc, out_hbm.at[cid])


_sc_hop = pl.kernel(
    _sc_hop_body,
    out_type=jax.ShapeDtypeStruct((NC, N), jnp.float32),
    mesh=_MESH,
    compiler_params=_SC_PARAMS,
    scratch_types=[
        pltpu.VMEM((N,), jnp.float32),
        pltpu.VMEM((N,), jnp.float32),
        pltpu.VMEM((N,), jnp.float32),
        pltpu.VMEM((16,), jnp.float32),
        pltpu.VMEM((RW, 128), jnp.int32),
        pltpu.VMEM((RW, 128), jnp.int32),
        pltpu.VMEM((RW, 128), jnp.float32),
        pltpu.VMEM_SHARED((N,), jnp.float32),
        pltpu.SemaphoreType.DMA,
    ],
)


def _pre_kernel(yh_ref, ys_ref, vin_ref, tout_ref, nsq_ref, nrm_ref):
    bq = yh_ref.shape[0]
    tout_ref[...] = jnp.sum((ys_ref[...] - yh_ref[...]), axis=0) / (
        bq * np.sqrt(D)
    )
    nsq = jnp.sum(vin_ref[...] * vin_ref[...], axis=1)
    nsq_ref[...] = nsq
    nrm_ref[...] = jnp.sqrt(nsq)


def _dense_kernel(vout_ref, vw_ref, good_ref, nsq_ref, nrm_ref, u2_ref,
                  tout_ref, out_ref):
    n = good_ref.shape[0]
    theta = THETA0 * (nsq_ref[...] + 1e-9)
    delta_g = good_ref[...] - theta
    f_prime = 1.0 - jnp.tanh(vw_ref[...]) ** 2
    rows = lax.broadcasted_iota(jnp.int32, (n,), 0)
    a = nrm_ref[...] * (u2_ref[0, :] + u2_ref[1, :]) + (rows == OUT_ID).astype(
        jnp.float32
    )
    t_rows = a[:, None] * tout_ref[...][None, :]
    dw = LAM_C * vout_ref[...] * f_prime * delta_g[:, None] + LAM_R * (
        t_rows * f_prime
    )
    g = jnp.sqrt(jnp.sum(dw * dw, axis=1, keepdims=True))
    out_ref[...] = jnp.where(g > 5.0, dw * (5.0 / (g + 1e-12)), dw)


def kernel(Y_hat, Y_star, V_in, V_out, V_weighted, goodness, spec_norm, edge_index):
    n = V_in.shape[0]
    tout, nsq, nrm = pl.pallas_call(
        _pre_kernel,
        out_shape=[
            jax.ShapeDtypeStruct((D,), jnp.float32),
            jax.ShapeDtypeStruct((n,), jnp.float32),
            jax.ShapeDtypeStruct((n,), jnp.float32),
        ],
    )(Y_hat, Y_star, V_in)

    pad = NW * RW * 128 - E
    pad_src = jnp.zeros((pad,), jnp.int32)
    pad_dst = (jnp.arange(pad, dtype=jnp.int32) * 37) % n
    src_p = jnp.concatenate([edge_index[0], pad_src]).reshape(NW, RW, 128)
    dst_p = jnp.concatenate([edge_index[1], pad_dst]).reshape(NW, RW, 128)
    edges = jnp.stack([src_p, dst_p])

    zvec = jnp.zeros((n,), jnp.float32)

    tot2 = _sc_total(nrm, edges, zvec)
    u2, pvec, gout = _sc_hop1(nrm, spec_norm, tot2, edges, zvec)
    for _ in range(HOPS - 1):
        u2 = _sc_hop(u2, pvec, gout, edges)

    dw = pl.pallas_call(
        _dense_kernel,
        out_shape=jax.ShapeDtypeStruct((n, D), jnp.float32),
    )(V_out, V_weighted, goodness, nsq, nrm, u2, tout)
    return dw


# speedup vs baseline: 92.6741x; 1.0015x over previous
import functools

import jax
import jax.numpy as jnp
import numpy as np
from jax import lax
from jax.experimental import pallas as pl
from jax.experimental.pallas import tpu as pltpu
from jax.experimental.pallas import tpu_sc as plsc


N = 10000
E = 320000
D = 128
LAM_C = 0.65
LAM_R = 0.35
THETA0 = 0.5
SPEC_MIN = 0.3
SPEC_MAX = 4.0
HOPS = 3
OUT_ID = N - 1

NC = 2
NS = 16
NW = NC * NS
EW = E // NW
RW = 80
RC = 16
NG = N // 16

_MESH = plsc.VectorSubcoreMesh(
    core_axis_name="c", subcore_axis_name="s", num_cores=NC, num_subcores=NS
)
_SC_PARAMS = pltpu.CompilerParams(needs_layout_passes=False)


def _worker_id():
    cid = lax.axis_index("c")
    sid = lax.axis_index("s")
    return cid, sid, cid * NS + sid


def _wait_all(descs):
    for d in descs:
        d.wait()


def _valid(j, k):
    return ((j < EW // 128) | ((j == EW // 128) & (k < (EW % 128) // 16))).astype(
        jnp.float32
    )


def _scatter_rows(compute_row, idx_t, val_t, acc, sem):
    def chunk(c, _):
        descs = []
        for i in range(RC):
            j = c * RC + i
            compute_row(j)
            descs.append(
                pltpu.async_copy(val_t.at[j], acc.at[idx_t.at[j]], sem, add=True)
            )
        _wait_all(descs)
        return 0

    lax.fori_loop(0, RW // RC, chunk, 0)


def _sc_total_body(nrm_hbm, edge_hbm, zvec_hbm, out_hbm,
                   nrm_loc, src_t, dst_t, val_t, acc, sem):
    cid, sid, wid = _worker_id()
    _wait_all([
        pltpu.async_copy(nrm_hbm, nrm_loc, sem),
        pltpu.async_copy(edge_hbm.at[0, wid], src_t, sem),
        pltpu.async_copy(edge_hbm.at[1, wid], dst_t, sem),
    ])

    @pl.when(sid == 0)
    def _():
        pltpu.sync_copy(zvec_hbm, acc)

    plsc.subcore_barrier()

    def compute_row(j):
        for k in range(8):
            s = pl.ds(k * 16, 16)
            v = plsc.load_gather(nrm_loc, [src_t[j, s]])
            val_t[j, s] = v * _valid(j, k)

    _scatter_rows(compute_row, dst_t, val_t, acc, sem)
    plsc.subcore_barrier()

    @pl.when(sid == 0)
    def _():
        pltpu.sync_copy(acc, out_hbm.at[cid])


_sc_total = pl.kernel(
    _sc_total_body,
    out_type=jax.ShapeDtypeStruct((NC, N), jnp.float32),
    mesh=_MESH,
    compiler_params=_SC_PARAMS,
    scratch_types=[
        pltpu.VMEM((N,), jnp.float32),
        pltpu.VMEM((RW, 128), jnp.int32),
        pltpu.VMEM((RW, 128), jnp.int32),
        pltpu.VMEM((RW, 128), jnp.float32),
        pltpu.VMEM_SHARED((N,), jnp.float32),
        pltpu.SemaphoreType.DMA,
    ],
)


def _sc_hop1_body(nrm_hbm, spec_hbm, tot2_hbm, edge_hbm, zvec_hbm,
                  u_hbm, p_hbm, gout_hbm,
                  spec_loc, ta_loc, tb_loc, nrm_loc, p_loc,
                  s8, ta8, tb8, g16, src_t, dst_t, val_t, acc, sem):
    cid, sid, wid = _worker_id()
    tile00 = (sid == 0) & (cid == 0)
    _wait_all([
        pltpu.async_copy(spec_hbm.at[pl.ds(N - 8, 8)], s8, sem),
        pltpu.async_copy(tot2_hbm.at[0, pl.ds(N - 8, 8)], ta8, sem),
        pltpu.async_copy(tot2_hbm.at[1, pl.ds(N - 8, 8)], tb8, sem),
        pltpu.async_copy(edge_hbm.at[0, wid], src_t, sem),
        pltpu.async_copy(edge_hbm.at[1, wid], dst_t, sem),
    ])

    @pl.when(sid == 0)
    def _():
        pltpu.sync_copy(zvec_hbm, acc)

    i7 = jnp.full((16,), 7, jnp.int32)
    sp = plsc.load_gather(s8, [i7])
    sp = jnp.minimum(jnp.maximum(sp, SPEC_MIN), SPEC_MAX)
    gout = sp / (plsc.load_gather(ta8, [i7]) + plsc.load_gather(tb8, [i7]) + 1e-9)

    @pl.when(tile00)
    def _():
        _wait_all([
            pltpu.async_copy(spec_hbm, spec_loc, sem),
            pltpu.async_copy(tot2_hbm.at[0], ta_loc, sem),
            pltpu.async_copy(tot2_hbm.at[1], tb_loc, sem),
            pltpu.async_copy(nrm_hbm, nrm_loc, sem),
        ])

        def pbody(i, _):
            s = pl.ds(i * 16, 16)
            spv = jnp.minimum(jnp.maximum(spec_loc[s], SPEC_MIN), SPEC_MAX)
            p_loc[s] = spv / (ta_loc[s] + tb_loc[s] + 1e-9) * nrm_loc[s]
            return 0

        lax.fori_loop(0, NG, pbody, 0)
        pltpu.sync_copy(p_loc, p_hbm)
        g16[...] = gout
        pltpu.sync_copy(g16, gout_hbm)

    plsc.subcore_barrier()
    out_id = jnp.int32(OUT_ID)

    def row_body(j, _):
        hits = jnp.zeros((16,), jnp.int32)
        for k in range(8):
            s = pl.ds(k * 16, 16)
            hit = (dst_t[j, s] == out_id).astype(jnp.int32)
            hits = hits + hit
            val_t[j, s] = gout * hit.astype(jnp.float32) * _valid(j, k)

        @pl.when(jnp.sum(hits) > 0)
        def _():
            pltpu.sync_copy(val_t.at[j], acc.at[src_t.at[j]], add=True)

        return 0

    lax.fori_loop(0, RW, row_body, 0)
    plsc.subcore_barrier()

    @pl.when(sid == 0)
    def _():
        pltpu.sync_copy(acc, u_hbm.at[cid])


_sc_hop1 = pl.kernel(
    _sc_hop1_body,
    out_type=[
        jax.ShapeDtypeStruct((NC, N), jnp.float32),
        jax.ShapeDtypeStruct((N,), jnp.float32),
        jax.ShapeDtypeStruct((16,), jnp.float32),
    ],
    mesh=_MESH,
    compiler_params=_SC_PARAMS,
    scratch_types=[
        pltpu.VMEM((N,), jnp.float32),
        pltpu.VMEM((N,), jnp.float32),
        pltpu.VMEM((N,), jnp.float32),
        pltpu.VMEM((N,), jnp.float32),
        pltpu.VMEM((N,), jnp.float32),
        pltpu.VMEM((8,), jnp.float32),
        pltpu.VMEM((8,), jnp.float32),
        pltpu.VMEM((8,), jnp.float32),
        pltpu.VMEM((16,), jnp.float32),
        pltpu.VMEM((RW, 128), jnp.int32),
        pltpu.VMEM((RW, 128), jnp.int32),
        pltpu.VMEM((RW, 128), jnp.float32),
        pltpu.VMEM_SHARED((N,), jnp.float32),
        pltpu.SemaphoreType.DMA,
    ],
)


def _sc_hop_body(u_hbm, p_hbm, gout_hbm, edge_hbm, out_hbm,
                 h_loc, ub_loc, p_loc, g16, src_t, dst_t, val_t, acc, sem):
    cid, sid, wid = _worker_id()
    _wait_all([
        pltpu.async_copy(u_hbm.at[0], h_loc, sem),
        pltpu.async_copy(u_hbm.at[1], ub_loc, sem),
        pltpu.async_copy(p_hbm, p_loc, sem),
        pltpu.async_copy(gout_hbm, g16, sem),
        pltpu.async_copy(edge_hbm.at[0, wid], src_t, sem),
        pltpu.async_copy(edge_hbm.at[1, wid], dst_t, sem),
    ])

    @pl.when(sid == 0)
    def _():
        pltpu.sync_copy(u_hbm.at[cid], acc)

    def hbody(i, _):
        s = pl.ds(i * 16, 16)
        h_loc[s] = p_loc[s] * (h_loc[s] + ub_loc[s])
        return 0

    lax.fori_loop(0, NG, hbody, 0)
    onehot = (lax.iota(jnp.int32, 16) == 15).astype(jnp.float32)
    lastg = pl.ds(16 * (NG - 1), 16)
    h_loc[lastg] = h_loc[lastg] + g16[...] * onehot

    plsc.subcore_barrier()

    def compute_row(j):
        for k in range(8):
            s = pl.ds(k * 16, 16)
            g = plsc.load_gather(h_loc, [dst_t[j, s]])
            val_t[j, s] = g * _valid(j, k)

    _scatter_rows(compute_row, src_t, val_t, acc, sem)
    plsc.subcore_barrier()

    @pl.when(sid == 0)
    def _():
        pltpu.sync_copy(acc, out_hbm.at[cid])


_sc_hop = pl.kernel(
    _sc_hop_body,
    out_type=jax.ShapeDtypeStruct((NC, N), jnp.float32),
    mesh=_MESH,
    compiler_params=_SC_PARAMS,
    scratch_types=[
        pltpu.VMEM((N,), jnp.float32),
        pltpu.VMEM((N,), jnp.float32),
        pltpu.VMEM((N,), jnp.float32),
        pltpu.VMEM((16,), jnp.float32),
        pltpu.VMEM((RW, 128), jnp.int32),
        pltpu.VMEM((RW, 128), jnp.int32),
        pltpu.VMEM((RW, 128), jnp.float32),
        pltpu.VMEM_SHARED((N,), jnp.float32),
        pltpu.SemaphoreType.DMA,
    ],
)


def _pre_kernel(yh_ref, ys_ref, vin_ref, tout_ref, nsq_ref, nrm_ref):
    bq = yh_ref.shape[0]
    tout_ref[...] = jnp.sum((ys_ref[...] - yh_ref[...]), axis=0) / (
        bq * np.sqrt(D)
    )
    nsq = jnp.sum(vin_ref[...] * vin_ref[...], axis=1)
    nsq_ref[...] = nsq
    nrm_ref[...] = jnp.sqrt(nsq)


def _dense_kernel(vout_ref, vw_ref, good_ref, nsq_ref, nrm_ref, u2_ref,
                  tout_ref, out_ref):
    n = good_ref.shape[0]
    theta = THETA0 * (nsq_ref[...] + 1e-9)
    delta_g = good_ref[...] - theta
    f_prime = 1.0 - jnp.tanh(vw_ref[...]) ** 2
    rows = lax.broadcasted_iota(jnp.int32, (n,), 0)
    a = nrm_ref[...] * (u2_ref[0, :] + u2_ref[1, :]) + (rows == OUT_ID).astype(
        jnp.float32
    )
    t_rows = a[:, None] * tout_ref[...][None, :]
    dw = LAM_C * vout_ref[...] * f_prime * delta_g[:, None] + LAM_R * (
        t_rows * f_prime
    )
    g = jnp.sqrt(jnp.sum(dw * dw, axis=1, keepdims=True))
    out_ref[...] = jnp.where(g > 5.0, dw * (5.0 / (g + 1e-12)), dw)


def kernel(Y_hat, Y_star, V_in, V_out, V_weighted, goodness, spec_norm, edge_index):
    n = V_in.shape[0]
    tout, nsq, nrm = pl.pallas_call(
        _pre_kernel,
        out_shape=[
            jax.ShapeDtypeStruct((D,), jnp.float32),
            jax.ShapeDtypeStruct((n,), jnp.float32),
            jax.ShapeDtypeStruct((n,), jnp.float32),
        ],
    )(Y_hat, Y_star, V_in)

    pad = NW * RW * 128 - E
    pad_src = jnp.zeros((pad,), jnp.int32)
    pad_dst = (jnp.arange(pad, dtype=jnp.int32) * 37) % n
    src_p = jnp.concatenate([edge_index[0], pad_src]).reshape(NW, RW, 128)
    dst_p = jnp.concatenate([edge_index[1], pad_dst]).reshape(NW, RW, 128)
    edges = jnp.stack([src_p, dst_p])

    zvec = jnp.zeros((n,), jnp.float32)

    tot2 = _sc_total(nrm, edges, zvec)
    u2, pvec, gout = _sc_hop1(nrm, spec_norm, tot2, edges, zvec)
    for _ in range(HOPS - 1):
        u2 = _sc_hop(u2, pvec, gout, edges)

    dw = pl.pallas_call(
        _dense_kernel,
        out_shape=jax.ShapeDtypeStruct((n, D), jnp.float32),
    )(V_out, V_weighted, goodness, nsq, nrm, u2, tout)
    return dw
